# baseline (speedup 1.0000x reference)
import functools

import jax
import jax.numpy as jnp
from jax import lax
from jax.experimental import pallas as pl
from jax.experimental.pallas import tpu as pltpu
from jax.experimental.pallas import tpu_sc as plsc

N = 10000
E = 320000
P = 100000
D = 128

NC = 2
NS = 16
NW = NC * NS
L = 16

CHUNK = 320
NPAD = CHUNK * NW
EW = E // NW
G = 80
C = 4000
NEG = -3.0e38

PW = 3136
PPAD = PW * NW

N_BLK = 1000
E_BLK = 2000


def _mm_body(x_ref, w_ref, o_ref):
    o_ref[...] = jnp.dot(x_ref[...], w_ref[...], preferred_element_type=jnp.float32)


def _mm(x, w, blk):
    n, d = x.shape
    k = w.shape[1]
    return pl.pallas_call(
        _mm_body,
        grid=(n // blk,),
        in_specs=[
            pl.BlockSpec((blk, d), lambda i: (i, 0)),
            pl.BlockSpec((d, k), lambda i: (0, 0)),
        ],
        out_specs=pl.BlockSpec((blk, k), lambda i: (i, 0)),
        out_shape=jax.ShapeDtypeStruct((n, k), jnp.float32),
    )(x, w)


def _lin_body(x_ref, seg_ref, pb_ref, wt_ref, wb_ref, lb_ref, o_ref):
    agg = jnp.maximum(seg_ref[...] + pb_ref[...], 0.0)
    h = jnp.dot(x_ref[...], wt_ref[...], preferred_element_type=jnp.float32)
    h = h + jnp.dot(agg, wb_ref[...], preferred_element_type=jnp.float32)
    o_ref[...] = jnp.maximum(h + lb_ref[...], 0.0)


def _fused_lin(x, seg, pool_b, w_top, w_bot, lin_b):
    n, d = x.shape
    k = w_top.shape[1]
    return pl.pallas_call(
        _lin_body,
        grid=(n // N_BLK,),
        in_specs=[
            pl.BlockSpec((N_BLK, d), lambda i: (i, 0)),
            pl.BlockSpec((N_BLK, d), lambda i: (i, 0)),
            pl.BlockSpec((1, d), lambda i: (0, 0)),
            pl.BlockSpec((d, k), lambda i: (0, 0)),
            pl.BlockSpec((d, k), lambda i: (0, 0)),
            pl.BlockSpec((1, k), lambda i: (0, 0)),
        ],
        out_specs=pl.BlockSpec((N_BLK, k), lambda i: (i, 0)),
        out_shape=jax.ShapeDtypeStruct((n, k), jnp.float32),
    )(x, seg, pool_b.reshape(1, d), w_top, w_bot, lin_b.reshape(1, k))


_MESH = plsc.VectorSubcoreMesh(core_axis_name="c", subcore_axis_name="s")


def _wid():
    return lax.axis_index("s") * NC + lax.axis_index("c")


@functools.partial(
    pl.kernel,
    mesh=_MESH,
    compiler_params=pltpu.CompilerParams(needs_layout_passes=False),
    out_type=jax.ShapeDtypeStruct((E, D), jnp.float32),
    scratch_types=[
        pltpu.VMEM((EW,), jnp.int32),
        pltpu.VMEM((EW,), jnp.float32),
        pltpu.VMEM((2, G, D), jnp.float32),
        pltpu.SemaphoreType.DMA,
        pltpu.SemaphoreType.DMA,
    ],
)
def _ef_kernel(x_hbm, src_hbm, scale_hbm, ef_hbm, src_v, sc_v, rows_v, sem0, sem1):
    base = _wid() * EW
    pltpu.sync_copy(src_hbm.at[pl.ds(base, EW)], src_v)
    pltpu.sync_copy(scale_hbm.at[pl.ds(base, EW)], sc_v)

    nb = EW // G

    def fire(b, buf, sem):
        pltpu.async_copy(x_hbm.at[src_v.at[pl.ds(b * G, G)]], rows_v.at[buf], sem)

    def drain(buf, sem):
        pltpu.make_async_copy(x_hbm.at[src_v.at[pl.ds(0, G)]], rows_v.at[buf], sem).wait()

    def process(b, buf):
        def body(gg, _):
            svec = sc_v[pl.ds(b * G + gg * L, L)]
            for i in range(L):
                s = svec[i]
                for j in range(D // L):
                    sl = pl.ds(j * L, L)
                    rows_v[buf, gg * L + i, sl] = rows_v[buf, gg * L + i, sl] * s
            return 0
        lax.fori_loop(0, G // L, body, 0)
        pltpu.sync_copy(rows_v.at[buf], ef_hbm.at[pl.ds(base + b * G, G)])

    fire(0, 0, sem0)

    def loop(k, _):
        fire(2 * k + 1, 1, sem1)
        drain(0, sem0)
        process(2 * k, 0)

        @pl.when(2 * k + 2 < nb)
        def _():
            fire(2 * k + 2, 0, sem0)

        drain(1, sem1)
        process(2 * k + 1, 1)
        return 0

    lax.fori_loop(0, nb // 2, loop, 0)
    drain(0, sem0)
    process(nb - 1, 0)


@functools.partial(
    pl.kernel,
    mesh=_MESH,
    compiler_params=pltpu.CompilerParams(needs_layout_passes=False),
    out_type=jax.ShapeDtypeStruct((NPAD, D), jnp.float32),
    scratch_types=[
        pltpu.VMEM((CHUNK + 1, D), jnp.float32),
        pltpu.VMEM((C,), jnp.int32),
        pltpu.VMEM((C + G,), jnp.int32),
        pltpu.VMEM((C + G,), jnp.int32),
        pltpu.VMEM((G, D), jnp.float32),
        pltpu.SemaphoreType.DMA,
    ],
)
def _segmax_kernel(val_hbm, dst_hbm, seg_hbm, acc, dstst, eidl, dstl, rows, sem):
    base = _wid() * CHUNK

    neg = jnp.full((L,), NEG, dtype=jnp.float32)

    def init(i, _):
        acc[i // (D // L), pl.ds((i % (D // L)) * L, L)] = neg
        return 0
    lax.fori_loop(0, (CHUNK + 1) * (D // L), init, 0)

    lanes = lax.iota(jnp.int32, L)
    trash = jnp.full((L,), CHUNK, dtype=jnp.int32)
    zeros = jnp.zeros((L,), dtype=jnp.int32)
    lstep = jnp.full((L,), L, dtype=jnp.int32)
    ones = jnp.ones((L,), dtype=jnp.int32)

    def chunk_body(cidx, _):
        pltpu.sync_copy(dst_hbm.at[pl.ds(cidx * C, C)], dstst)
        blo = jnp.full((L,), base, jnp.int32)
        bhi = jnp.full((L,), base + CHUNK, jnp.int32)
        eid0 = jnp.full((L,), cidx * C, jnp.int32) + lanes

        def scan(g, carry):
            o, eid = carry
            dv = dstst[pl.ds(g * L, L)]
            m = (dv >= blo) & (dv < bhi)
            cnt = plsc.cumsum(jnp.where(m, ones, zeros))[L - 1]
            plsc.store_compressed(eidl.at[pl.ds(o, L)], eid, mask=m)
            plsc.store_compressed(dstl.at[pl.ds(o, L)], dv - blo, mask=m)
            return (o + cnt, eid + lstep)
        o, _unused = lax.fori_loop(0, C // L, scan, (0, eid0))

        def pad(t, _):
            eidl[pl.ds(o + t * L, L)] = zeros
            dstl[pl.ds(o + t * L, L)] = trash
            return 0
        lax.fori_loop(0, G // L, pad, 0)
        nb = (o + G - 1) // G

        def batch(q, _):
            pltpu.async_copy(
                val_hbm.at[eidl.at[pl.ds(q * G, G)]], rows, sem).wait()

            def upd(gg, _):
                rvec = dstl[pl.ds(q * G + gg * L, L)]
                for i in range(L):
                    r = rvec[i]
                    for j in range(D // L):
                        sl = pl.ds(j * L, L)
                        acc[r, sl] = jnp.maximum(acc[r, sl], rows[gg * L + i, sl])
                return 0
            lax.fori_loop(0, G // L, upd, 0)
            return 0
        lax.fori_loop(0, nb, batch, 0)
        return 0

    lax.fori_loop(0, E // C, chunk_body, 0)
    pltpu.sync_copy(acc.at[pl.ds(0, CHUNK)], seg_hbm.at[pl.ds(base, CHUNK)])


@functools.partial(
    pl.kernel,
    mesh=_MESH,
    compiler_params=pltpu.CompilerParams(needs_layout_passes=False),
    out_type=(
        jax.ShapeDtypeStruct((PPAD,), jnp.float32),
        jax.ShapeDtypeStruct((PPAD,), jnp.float32),
    ),
    scratch_types=[
        pltpu.VMEM((N,), jnp.float32),
        pltpu.VMEM((N,), jnp.float32),
        pltpu.VMEM((PW,), jnp.int32),
        pltpu.VMEM((PW,), jnp.int32),
        pltpu.VMEM((PW,), jnp.float32),
        pltpu.VMEM((PW,), jnp.float32),
        pltpu.VMEM((2 * L,), jnp.float32),
    ],
)
def _edge_head_kernel(zw_hbm, zp_hbm, pe0_hbm, pe1_hbm, bias_hbm, ew_hbm, ep_hbm,
                      zwtab, zptab, p0, p1, ewv, epv, bv):
    base = _wid() * PW
    pltpu.sync_copy(zw_hbm, zwtab)
    pltpu.sync_copy(zp_hbm, zptab)
    pltpu.sync_copy(pe0_hbm.at[pl.ds(base, PW)], p0)
    pltpu.sync_copy(pe1_hbm.at[pl.ds(base, PW)], p1)
    pltpu.sync_copy(bias_hbm, bv)
    bw = bv[pl.ds(0, L)]
    bp = bv[pl.ds(L, L)]

    def body(g, _):
        sl = pl.ds(g * L, L)
        i0 = p0[sl]
        i1 = p1[sl]
        z0w = plsc.load_gather(zwtab, [i0])
        z1w = plsc.load_gather(zwtab, [i1])
        z0p = plsc.load_gather(zptab, [i0])
        z1p = plsc.load_gather(zptab, [i1])
        ewv[sl] = jnp.maximum(z0w + z1w + bw, 0.0)
        epv[sl] = z0p + z1p + bp
        return 0
    lax.fori_loop(0, PW // L, body, 0)

    pltpu.sync_copy(ewv, ew_hbm.at[pl.ds(base, PW)])
    pltpu.sync_copy(epv, ep_hbm.at[pl.ds(base, PW)])


def _conv(x, src, dst, scale, pool_w, pool_b, lin_w, lin_b):
    ef = _ef_kernel(x, src, scale)
    pooledraw = _mm(ef, pool_w, E_BLK)
    seg = _segmax_kernel(pooledraw, dst)[:N]
    d = x.shape[1]
    return _fused_lin(x, seg, pool_b, lin_w[:d], lin_w[d:], lin_b)


def kernel(x, prediction_edges, message_edges, message_edgewt,
           pool1_w, pool1_b, coef1, lin1_w, lin1_b,
           pool2_w, pool2_b, coef2, lin2_w, lin2_b,
           ewp_w, ewp_b, ep_w, ep_b):
    src, dst = message_edges[0], message_edges[1]

    scale1 = 1.0 + coef1 * message_edgewt
    h1 = _conv(x, src, dst, scale1, pool1_w, pool1_b, lin1_w, lin1_b)

    scale2 = 1.0 + coef2 * message_edgewt
    h2 = _conv(h1, src, dst, scale2, pool2_w, pool2_b, lin2_w, lin2_b)

    wz = jnp.zeros((D, 128), jnp.float32).at[:, 0:1].set(ewp_w).at[:, 1:2].set(ep_w)
    z = _mm(h2, wz, N_BLK)
    zw, zp = z[:, 0], z[:, 1]

    pe0 = jnp.zeros((PPAD,), jnp.int32).at[:P].set(prediction_edges[0])
    pe1 = jnp.zeros((PPAD,), jnp.int32).at[:P].set(prediction_edges[1])
    bias = jnp.concatenate([
        jnp.full((L,), ewp_b[0], jnp.float32),
        jnp.full((L,), ep_b[0], jnp.float32),
    ])
    ew, ep = _edge_head_kernel(zw, zp, pe0, pe1, bias)
    return (ew[:P].reshape(P, 1), ep[:P].reshape(P, 1))
